# Initial kernel scaffold; baseline (speedup 1.0000x reference)
import jax
import jax.numpy as jnp
from jax.experimental import pallas as pl


def kernel(a, delta_a, b, delta_b, history, delta_history, ffm_embs, linear_w, linear_bias):
    raise NotImplementedError("write your pallas kernel here")


# speedup vs baseline: 16.5814x; 16.5814x over previous
import functools

import numpy as np
import jax
import jax.numpy as jnp
from jax import lax
from jax.experimental import pallas as pl
from jax.experimental.pallas import tpu as pltpu
from jax.experimental.pallas import tpu_sc as plsc

N_PAGES = 50000
N_DELTAS = 64
HIST = 10
EMB = 16
B = 4096
_FIELD_DIMS = [N_PAGES, N_DELTAS, N_PAGES, N_DELTAS] + [N_PAGES] * HIST + [N_DELTAS] * HIST
F = len(_FIELD_DIMS)
TOTAL = int(sum(_FIELD_DIMS))
_OFFS = np.concatenate(([0], np.cumsum(_FIELD_DIMS)[:-1])).astype(np.int32)

_I_ARR, _J_ARR = np.triu_indices(F, k=1)
NPAIR = _I_ARR.shape[0]

NC = 2
NS = 16
NW = NC * NS
SPT = B // NW
CH = 4
NCH = SPT // CH
IPC = CH * F
NG = F // 8
RPC = NG * IPC
LPS = 32
LPT = SPT * LPS
IDXW = 128
NLG = LPT // IDXW

_PAR = ((_I_ARR // 8) * IPC + _J_ARR).tolist()
_PAO = ((_I_ARR % 8) * EMB).tolist()
_PBR = ((_J_ARR // 8) * IPC + _I_ARR).tolist()
_PBO = ((_J_ARR % 8) * EMB).tolist()


@functools.lru_cache(maxsize=1)
def _build_ffm_sc():
    mesh = plsc.VectorSubcoreMesh(core_axis_name="c", subcore_axis_name="s")
    return functools.partial(
        pl.kernel,
        mesh=mesh,
        out_type=jax.ShapeDtypeStruct((B,), jnp.float32),
        scratch_types=[
            pltpu.VMEM((IPC,), jnp.int32),
            pltpu.VMEM((IPC,), jnp.int32),
            pltpu.VMEM((RPC, 128), jnp.float32),
            pltpu.VMEM((RPC, 128), jnp.float32),
            pltpu.VMEM((NLG, IDXW), jnp.int32),
            pltpu.VMEM((LPT,), jnp.float32),
            pltpu.VMEM((SPT,), jnp.float32),
            pltpu.SemaphoreType.DMA,
            pltpu.SemaphoreType.DMA,
            pltpu.SemaphoreType.DMA,
        ],
        compiler_params=pltpu.CompilerParams(use_tc_tiling_on_sc=False),
    )(_ffm_sc)


def _ffm_sc(packed_h, xi_h, lidx_h, wpad_h, out_h,
            idx0, idx1, rows0, rows1, lidx_v, lw_v, out_v,
            sem0, sem1, lsem):
    wid = lax.axis_index("s") * NC + lax.axis_index("c")
    s0 = wid * SPT
    idx_bufs = (idx0, idx1)
    row_bufs = (rows0, rows1)
    sems = (sem0, sem1)

    def fetch_chunk(c, bslot):
        idx_b = idx_bufs[bslot]
        rows_b = row_bufs[bslot]
        pltpu.sync_copy(xi_h.at[pl.ds((s0 + c * CH) * F, IPC)], idx_b)
        for g in range(NG):
            pltpu.async_copy(packed_h.at[g].at[idx_b],
                             rows_b.at[pl.ds(g * IPC, IPC)], sems[bslot])

    def wait_chunk(bslot):
        pltpu.make_async_copy(packed_h.at[0].at[pl.ds(0, RPC)],
                              row_bufs[bslot], sems[bslot]).wait()

    pltpu.sync_copy(lidx_h.at[pl.ds(wid * NLG, NLG)], lidx_v)
    for k in range(NLG):
        pltpu.async_copy(wpad_h.at[lidx_v.at[k]],
                         lw_v.at[pl.ds(k * IDXW, IDXW)], lsem)

    fetch_chunk(0, 0)
    fetch_chunk(1, 1)
    pltpu.make_async_copy(wpad_h.at[pl.ds(0, LPT)], lw_v, lsem).wait()

    lane_iota = lax.iota(jnp.int32, 16)

    def lane_sum_all(v):
        for m in (8, 4, 2, 1):
            p = jnp.bitwise_xor(lane_iota, m)
            v = v + v.at[p].get(mode="promise_in_bounds")
        return v

    def outer(q, carry):
        vec = jnp.zeros((16,), jnp.float32)
        for t in range(4):
            c = q * 4 + t
            bslot = t % 2
            wait_chunk(bslot)
            rows_b = row_bufs[bslot]

            def sample_body(s, vec, rows_b=rows_b, c=c, t=t):
                sr = s * F
                acc = jnp.zeros((EMB,), jnp.float32)
                for p in range(NPAIR):
                    va = rows_b[sr + _PAR[p], pl.ds(_PAO[p], EMB)]
                    vb = rows_b[sr + _PBR[p], pl.ds(_PBO[p], EMB)]
                    acc = acc + va * vb
                gl = c * CH + s
                lw0 = lw_v[pl.ds(gl * LPS, 16)]
                lw1 = lw_v[pl.ds(gl * LPS + 16, 16)]
                tot = lane_sum_all(acc + lw0 + lw1)
                lane = t * CH + s
                return jnp.where(lane_iota == lane, tot, vec)

            vec = lax.fori_loop(0, CH, sample_body, vec)

            @pl.when(c + 2 < NCH)
            def _():
                fetch_chunk(c + 2, bslot)
        out_v[pl.ds(q * 16, 16)] = 1.0 / (1.0 + jnp.exp(-vec))
        return carry

    lax.fori_loop(0, NCH // 4, outer, 0)
    pltpu.sync_copy(out_v, out_h.at[pl.ds(s0, SPT)])


def kernel(a, delta_a, b, delta_b, history, delta_history, ffm_embs, linear_w, linear_bias):
    offs = jnp.asarray(_OFFS)
    x = jnp.concatenate(
        [a[:, None], delta_a[:, None], b[:, None], delta_b[:, None],
         history, delta_history], axis=1).astype(jnp.int32)
    xi = x + offs[None, :]

    lidx = jnp.concatenate(
        [xi,
         jnp.full((B, 1), TOTAL, jnp.int32),
         jnp.full((B, 7), TOTAL + 1, jnp.int32)],
        axis=1).reshape(-1, IDXW)
    wpad = jnp.concatenate(
        [linear_w.reshape(-1), linear_bias.astype(jnp.float32),
         jnp.zeros((7,), jnp.float32)])

    packed = jnp.transpose(ffm_embs.reshape(NG, 8, TOTAL, EMB),
                           (0, 2, 1, 3)).reshape(NG, TOTAL, 8 * EMB)

    return _build_ffm_sc()(packed, xi.reshape(-1), lidx, wpad)
